# baseline (speedup 1.0000x reference)
import functools

import jax
import jax.numpy as jnp
from jax.experimental import pallas as pl
from jax.experimental.pallas import tpu as pltpu

_DIM = 1024
_NUM_LAYERS = 6
_STRENGTHS = (0.1, 0.1, 0.05)
_TILE = 1024


def _row_normalize(x):
    n = jnp.sqrt(jnp.sum(x * x, axis=-1, keepdims=True))
    return x / jnp.maximum(n, 1e-12)


def _collapse_kernel(h_ref, w1_ref, b1_ref, w2_ref, b2_ref, anc_ref, out_ref,
                     *, precision):
    h = h_ref[:]
    w1 = w1_ref[:]
    w2 = w2_ref[:]
    b1 = b1_ref[:]
    b2 = b2_ref[:]
    anchors = _row_normalize(anc_ref[:])

    dn = (((1,), (1,)), ((), ()))
    for _ in range(_NUM_LAYERS):
        h_n = _row_normalize(h)
        hidden = jnp.tanh(
            jax.lax.dot_general(h, w1, dn, precision=precision,
                                preferred_element_type=jnp.float32) + b1)
        delta = jax.lax.dot_general(hidden, w2, dn, precision=precision,
                                    preferred_element_type=jnp.float32) + b2
        force = jnp.zeros_like(h)
        for i in range(3):
            a = anchors[i:i + 1, :]
            align = jnp.sum(h_n * a, axis=-1, keepdims=True)
            div = 1.0 - align
            direction = _row_normalize(h - a)
            force = force + _STRENGTHS[i] * div * direction
        h = h + delta - force
        h_norm = jnp.sqrt(jnp.sum(h * h, axis=-1, keepdims=True))
        h = jnp.where(h_norm > 10.0, h * (10.0 / (h_norm + 1e-08)), h)
    out_ref[:] = h


def kernel(h0, W1, b1, W2, b2, anchor_entail, anchor_contra, anchor_neutral):
    squeeze = h0.ndim == 1
    h = h0[None, :] if squeeze else h0
    n = h.shape[0]
    tile = _TILE if n % _TILE == 0 else n
    anchors = jnp.stack([anchor_entail, anchor_contra, anchor_neutral], axis=0)
    b1_2d = b1[None, :]
    b2_2d = b2[None, :]

    out = pl.pallas_call(
        functools.partial(_collapse_kernel, precision=jax.lax.Precision.DEFAULT),
        grid=(n // tile,),
        in_specs=[
            pl.BlockSpec((tile, _DIM), lambda i: (i, 0)),
            pl.BlockSpec((_DIM, _DIM), lambda i: (0, 0)),
            pl.BlockSpec((1, _DIM), lambda i: (0, 0)),
            pl.BlockSpec((_DIM, _DIM), lambda i: (0, 0)),
            pl.BlockSpec((1, _DIM), lambda i: (0, 0)),
            pl.BlockSpec((3, _DIM), lambda i: (0, 0)),
        ],
        out_specs=pl.BlockSpec((tile, _DIM), lambda i: (i, 0)),
        out_shape=jax.ShapeDtypeStruct((n, _DIM), jnp.float32),
        compiler_params=pltpu.CompilerParams(
            dimension_semantics=("parallel",),
        ),
    )(h, W1, b1_2d, W2, b2_2d, anchors)
    return out[0] if squeeze else out


# speedup vs baseline: 1.3054x; 1.3054x over previous
import functools

import jax
import jax.numpy as jnp
from jax.experimental import pallas as pl
from jax.experimental.pallas import tpu as pltpu

_DIM = 1024
_NUM_LAYERS = 6
_STRENGTHS = (0.1, 0.1, 0.05)
_TILE = 256


def _row_normalize(x):
    n = jnp.sqrt(jnp.sum(x * x, axis=-1, keepdims=True))
    return x / jnp.maximum(n, 1e-12)


def _collapse_kernel(h_ref, w1_ref, b1_ref, w2_ref, b2_ref, anc_ref, out_ref,
                     *, precision):
    h = h_ref[:]
    w1 = w1_ref[:]
    w2 = w2_ref[:]
    b1 = b1_ref[:]
    b2 = b2_ref[:]
    anchors = _row_normalize(anc_ref[:])

    dn = (((1,), (1,)), ((), ()))
    for _ in range(_NUM_LAYERS):
        h_n = _row_normalize(h)
        hidden = jnp.tanh(
            jax.lax.dot_general(h, w1, dn, precision=precision,
                                preferred_element_type=jnp.float32) + b1)
        delta = jax.lax.dot_general(hidden, w2, dn, precision=precision,
                                    preferred_element_type=jnp.float32) + b2
        force = jnp.zeros_like(h)
        for i in range(3):
            a = anchors[i:i + 1, :]
            align = jnp.sum(h_n * a, axis=-1, keepdims=True)
            div = 1.0 - align
            direction = _row_normalize(h - a)
            force = force + _STRENGTHS[i] * div * direction
        h = h + delta - force
        h_norm = jnp.sqrt(jnp.sum(h * h, axis=-1, keepdims=True))
        h = jnp.where(h_norm > 10.0, h * (10.0 / (h_norm + 1e-08)), h)
    out_ref[:] = h


def kernel(h0, W1, b1, W2, b2, anchor_entail, anchor_contra, anchor_neutral):
    squeeze = h0.ndim == 1
    h = h0[None, :] if squeeze else h0
    n = h.shape[0]
    tile = _TILE if n % _TILE == 0 else n
    anchors = jnp.stack([anchor_entail, anchor_contra, anchor_neutral], axis=0)
    b1_2d = b1[None, :]
    b2_2d = b2[None, :]

    out = pl.pallas_call(
        functools.partial(_collapse_kernel, precision=jax.lax.Precision.DEFAULT),
        grid=(n // tile,),
        in_specs=[
            pl.BlockSpec((tile, _DIM), lambda i: (i, 0)),
            pl.BlockSpec((_DIM, _DIM), lambda i: (0, 0)),
            pl.BlockSpec((1, _DIM), lambda i: (0, 0)),
            pl.BlockSpec((_DIM, _DIM), lambda i: (0, 0)),
            pl.BlockSpec((1, _DIM), lambda i: (0, 0)),
            pl.BlockSpec((3, _DIM), lambda i: (0, 0)),
        ],
        out_specs=pl.BlockSpec((tile, _DIM), lambda i: (i, 0)),
        out_shape=jax.ShapeDtypeStruct((n, _DIM), jnp.float32),
        compiler_params=pltpu.CompilerParams(
            dimension_semantics=("parallel",),
        ),
    )(h, W1, b1_2d, W2, b2_2d, anchors)
    return out[0] if squeeze else out
